# Initial kernel scaffold; baseline (speedup 1.0000x reference)
import jax
import jax.numpy as jnp
from jax.experimental import pallas as pl


def kernel(used_symbols, distribution, symbol_embeddings, positional_embeddings):
    raise NotImplementedError("write your pallas kernel here")


# speedup vs baseline: 7.9236x; 7.9236x over previous
import functools

import jax
import jax.numpy as jnp
from jax import lax
from jax.experimental import pallas as pl
from jax.experimental.pallas import tpu as pltpu
from jax.experimental.pallas import tpu_sc as plsc

NC = 2
NS = 16
NW = NC * NS

CHUNK = 1024
IDX_MINOR = 128


def _combine_body(sym_ref, pos_ref, t_ref):
    t_ref[...] = pos_ref[...] + sym_ref[...]


def _combine_tables(sym_padded, pos):
    return pl.pallas_call(
        _combine_body,
        out_shape=jax.ShapeDtypeStruct(pos.shape, pos.dtype),
    )(sym_padded, pos)


def _sc_gather(table, idx2, dist2, n_total, d):
    n_per = n_total // NW
    n_chunks = n_per // CHUNK
    g_per_chunk = CHUNK // IDX_MINOR

    mesh = plsc.VectorSubcoreMesh(
        core_axis_name="c", subcore_axis_name="s", num_cores=NC,
        num_subcores=NS)

    @functools.partial(
        pl.kernel,
        out_type=jax.ShapeDtypeStruct((n_total, d), jnp.float32),
        mesh=mesh,
        compiler_params=pltpu.CompilerParams(needs_layout_passes=False,
                                             use_tc_tiling_on_sc=False),
        scratch_types=[
            pltpu.VMEM((g_per_chunk, IDX_MINOR), jnp.int32),
            pltpu.VMEM((CHUNK // 16, 16), jnp.float32),
            pltpu.VMEM((CHUNK, d), jnp.float32),
            pltpu.SemaphoreType.DMA,
        ],
    )
    def run(t_hbm, idx_hbm, dist_hbm, out_hbm, idx_v, dist_v, rows_v, sem):
        wid = lax.axis_index("s") * NC + lax.axis_index("c")
        base = wid * n_per

        def chunk_body(c, _):
            off = base + c * CHUNK
            irow = pl.multiple_of(off // IDX_MINOR, g_per_chunk)
            drow = pl.multiple_of(off // 16, CHUNK // 16)
            pltpu.sync_copy(idx_hbm.at[pl.ds(irow, g_per_chunk)], idx_v)
            pltpu.sync_copy(dist_hbm.at[pl.ds(drow, CHUNK // 16)], dist_v)
            copies = []
            for j in range(g_per_chunk):
                copies.append(pltpu.async_copy(
                    t_hbm.at[idx_v.at[j]],
                    rows_v.at[pl.ds(j * IDX_MINOR, IDX_MINOR)],
                    sem))
            for cp in copies:
                cp.wait()

            col_ids = jnp.full((16,), d - 1, jnp.int32)
            lane = lax.iota(jnp.int32, 16)

            def fix_body(i, _):
                row_ids = i * 16 + lane
                plsc.addupdate_scatter(rows_v, [row_ids, col_ids], dist_v[i])
                return 0

            lax.fori_loop(0, CHUNK // 16, fix_body, 0)
            pltpu.sync_copy(rows_v, out_hbm.at[pl.ds(off, CHUNK)])
            return 0

        lax.fori_loop(0, n_chunks, chunk_body, 0)

    return run(table, idx2, dist2)


def kernel(used_symbols, distribution, symbol_embeddings, positional_embeddings):
    b, l = used_symbols.shape
    v, dm1 = symbol_embeddings.shape
    d = dm1 + 1
    n = b * l

    sym_padded = jnp.pad(symbol_embeddings, ((0, 0), (0, 1)))
    table = _combine_tables(sym_padded, positional_embeddings)

    idx2 = used_symbols.astype(jnp.int32).reshape(n // IDX_MINOR, IDX_MINOR)
    dist2 = distribution.reshape(n // 16, 16)
    out = _sc_gather(table, idx2, dist2, n, d)
    return out.reshape(b, l, d)
